# baseline (speedup 1.0000x reference)
import functools

import jax
import jax.numpy as jnp
from jax import lax
from jax.experimental import pallas as pl
from jax.experimental.pallas import tpu as pltpu
from jax.experimental.pallas import tpu_sc as plsc

BATCH = 4096
HIST = 200
EMBED = 64
VOCAB = 1000000
N = BATCH * HIST
NC, NS = 2, 16
NW = NC * NS

UNITS = VOCAB // 128
UNITS_BASE = UNITS // NW
UNITS_EXTRA = UNITS % NW

PER_W = N // NW
CH = 320
NCHUNK = PER_W // CH
PAIRS = NCHUNK // 2


def _sc_transpose(tT, tail):
    mesh = plsc.VectorSubcoreMesh(core_axis_name="c", subcore_axis_name="s")

    @functools.partial(
        pl.kernel,
        out_type=jax.ShapeDtypeStruct((VOCAB, 128), jnp.float32),
        mesh=mesh,
        scratch_types=[
            pltpu.VMEM((EMBED, 128), jnp.float32),
            pltpu.VMEM((EMBED, 128), jnp.float32),
            pltpu.VMEM((128, 129), jnp.float32),
            pltpu.VMEM((128, 129), jnp.float32),
            pltpu.SemaphoreType.DMA,
            pltpu.SemaphoreType.DMA,
            pltpu.SemaphoreType.DMA,
            pltpu.SemaphoreType.DMA,
        ],
        compiler_params=pltpu.CompilerParams(needs_layout_passes=False),
    )
    def k(tT_hbm, tail_hbm, out_hbm, in0, in1, tp0, tp1, rs0, rs1, ws0, ws1):
        wid = lax.axis_index("s") * NC + lax.axis_index("c")
        base = UNITS_BASE * wid + jnp.minimum(wid, UNITS_EXTRA)
        cnt_full = UNITS_BASE + jnp.where(wid < UNITS_EXTRA, 1, 0)
        in_v = (in0, in1)
        tp_v = (tp0, tp1)
        rsem = (rs0, rs1)
        wsem = (ws0, ws1)

        col_ids = [
            jax.lax.iota(jnp.int32, 16) + c0 * 16
            for c0 in range(EMBED // 16)
        ]
        row_ids = [
            jax.lax.iota(jnp.int32, 16) + r0 * 16 for r0 in range(8)
        ]

        def start_read(u, b, w):
            pltpu.async_copy(
                tT_hbm.at[:, pl.ds(u * 128, w)],
                in_v[b].at[:, pl.ds(0, w)], rsem[b])

        def wait_read(b, w):
            pltpu.make_async_copy(
                tT_hbm.at[:, pl.ds(0, w)],
                in_v[b].at[:, pl.ds(0, w)], rsem[b]).wait()

        def transpose(b, w):
            @plsc.parallel_loop(0, EMBED, step=2, unroll=4)
            def _(c0):
                for dc in range(2):
                    c = c0 + dc
                    cs = jnp.zeros((16,), jnp.int32) + c
                    for r0 in range(w // 16):
                        x = in_v[b][c, pl.ds(r0 * 16, 16)]
                        plsc.store_scatter(tp_v[b], (row_ids[r0], cs), x)

        def start_write(u, b, w):
            pltpu.async_copy(
                tp_v[b].at[pl.ds(0, w), pl.ds(0, 128)],
                out_hbm.at[pl.ds(u * 128, w)], wsem[b])

        def wait_write(b, w):
            pltpu.make_async_copy(
                tp_v[b].at[pl.ds(0, w), pl.ds(0, 128)],
                out_hbm.at[pl.ds(0, w)], wsem[b]).wait()

        n_pairs = cnt_full // 2
        has_tail = cnt_full - 2 * n_pairs
        last = base + cnt_full - 1

        start_read(base, 0, 128)
        start_read(base + 1, 1, 128)

        wait_read(0, 128)
        transpose(0, 128)
        start_read(base + 2, 0, 128)
        start_write(base, 0, 128)

        wait_read(1, 128)
        transpose(1, 128)
        start_read(base + 3, 1, 128)
        start_write(base + 1, 1, 128)

        def pair_fn(p, carry):
            u0 = base + 2 * p
            wait_read(0, 128)
            wait_write(0, 128)
            transpose(0, 128)
            start_read(jnp.minimum(u0 + 2, last), 0, 128)
            start_write(u0, 0, 128)

            wait_read(1, 128)
            wait_write(1, 128)
            transpose(1, 128)
            start_read(jnp.minimum(u0 + 3, last), 1, 128)
            start_write(u0 + 1, 1, 128)
            return carry

        lax.fori_loop(1, n_pairs, pair_fn, 0)

        @pl.when(has_tail == 1)
        def _():
            wait_read(0, 128)
            wait_write(0, 128)
            transpose(0, 128)
            start_write(last, 0, 128)
            wait_write(0, 128)

        @pl.when(has_tail == 0)
        def _():
            wait_read(0, 128)
            wait_write(0, 128)

        wait_read(1, 128)
        wait_write(1, 128)

        @pl.when(wid == NW - 1)
        def _():
            pltpu.sync_copy(tail_hbm, in_v[0])
            pltpu.sync_copy(in_v[0],
                            out_hbm.at[pl.ds(UNITS * 128, EMBED)])

    return k(tT, tail)


def _sc_gather(idx_flat, table128):
    mesh = plsc.VectorSubcoreMesh(core_axis_name="c", subcore_axis_name="s")

    @functools.partial(
        pl.kernel,
        out_type=jax.ShapeDtypeStruct((N, 128), jnp.float32),
        mesh=mesh,
        scratch_types=[
            pltpu.VMEM((CH,), jnp.int32),
            pltpu.VMEM((CH,), jnp.int32),
            pltpu.VMEM((CH, 128), jnp.float32),
            pltpu.VMEM((CH, 128), jnp.float32),
            pltpu.SemaphoreType.DMA,
            pltpu.SemaphoreType.DMA,
            pltpu.SemaphoreType.DMA,
            pltpu.SemaphoreType.DMA,
            pltpu.SemaphoreType.DMA,
            pltpu.SemaphoreType.DMA,
        ],
    )
    def k(idx_hbm, table_hbm, out_hbm, idx0, idx1, rows0, rows1,
          isem0, isem1, gsem0, gsem1, wsem0, wsem1):
        wid = lax.axis_index("s") * NC + lax.axis_index("c")
        base = wid * PER_W
        idx_v = (idx0, idx1)
        rows_v = (rows0, rows1)
        isem = (isem0, isem1)
        gsem = (gsem0, gsem1)
        wsem = (wsem0, wsem1)

        def start_idx(c, b):
            pltpu.async_copy(
                idx_hbm.at[pl.ds(base + c * CH, CH)], idx_v[b], isem[b])

        def wait_idx(b):
            pltpu.make_async_copy(
                idx_hbm.at[pl.ds(0, CH)], idx_v[b], isem[b]).wait()

        def start_gather(b):
            pltpu.async_copy(table_hbm.at[idx_v[b]], rows_v[b], gsem[b])

        def wait_gather(b):
            pltpu.make_async_copy(
                table_hbm.at[pl.ds(0, CH)], rows_v[b], gsem[b]).wait()

        def start_wb(c, b):
            pltpu.async_copy(
                rows_v[b], out_hbm.at[pl.ds(base + c * CH, CH)], wsem[b])

        def wait_wb(b):
            pltpu.make_async_copy(
                rows_v[b], out_hbm.at[pl.ds(0, CH)], wsem[b]).wait()

        start_idx(0, 0)
        start_idx(1, 1)
        wait_idx(0)
        start_gather(0)

        wait_gather(0)
        start_wb(0, 0)
        start_idx(2, 0)
        wait_idx(1)
        start_gather(1)

        wait_gather(1)
        start_wb(1, 1)
        start_idx(3, 1)
        wait_wb(0)
        wait_idx(0)
        start_gather(0)

        def pair_body(p, carry):
            c0 = 2 * p
            wait_gather(0)
            start_wb(c0, 0)
            start_idx(c0 + 2, 0)
            wait_wb(1)
            wait_idx(1)
            start_gather(1)

            wait_gather(1)
            start_wb(c0 + 1, 1)
            start_idx(c0 + 3, 1)
            wait_wb(0)
            wait_idx(0)
            start_gather(0)
            return carry

        lax.fori_loop(1, PAIRS - 1, pair_body, 0)

        wait_gather(0)
        start_wb(NCHUNK - 2, 0)
        wait_wb(1)
        wait_idx(1)
        start_gather(1)

        wait_gather(1)
        start_wb(NCHUNK - 1, 1)

        wait_wb(0)
        wait_wb(1)

    return k(idx_flat, table128)


def _tail_body(tT_ref, o_ref):
    x = tT_ref[...]
    t = x.T
    o_ref[...] = jnp.concatenate(
        [t[:EMBED, :], jnp.zeros((EMBED, 128 - EMBED), jnp.float32)], axis=1)


def _tc_tail(tT):
    return pl.pallas_call(
        _tail_body,
        out_shape=jax.ShapeDtypeStruct((EMBED, 128), jnp.float32),
        in_specs=[pl.BlockSpec((EMBED, 128), lambda i: (0, UNITS))],
        out_specs=pl.BlockSpec((EMBED, 128), lambda i: (0, 0)),
        grid=(1,),
    )(tT)


def _mask_body(t_ref, m_ref):
    m_ref[...] = t_ref[...] == 0


def _tc_mask(term):
    blk = 256
    return pl.pallas_call(
        _mask_body,
        out_shape=jax.ShapeDtypeStruct((BATCH, HIST), jnp.bool_),
        in_specs=[pl.BlockSpec((blk, HIST), lambda i: (i, 0))],
        out_specs=pl.BlockSpec((blk, HIST), lambda i: (i, 0)),
        grid=(BATCH // blk,),
    )(term)


def kernel(term, table):
    idx_flat = term.reshape(N)
    tT = table.T
    table128 = _sc_transpose(tT, _tc_tail(tT))
    rows = _sc_gather(idx_flat, table128)
    embedded = rows[:, :EMBED].reshape(BATCH, HIST, EMBED)
    mask = _tc_mask(term)
    return (embedded, mask)


# speedup vs baseline: 1.6352x; 1.6352x over previous
import functools

import jax
import jax.numpy as jnp
from jax import lax
from jax.experimental import pallas as pl
from jax.experimental.pallas import tpu as pltpu
from jax.experimental.pallas import tpu_sc as plsc

BATCH = 4096
HIST = 200
EMBED = 64
N = BATCH * HIST
NC, NS = 2, 16
NW = NC * NS
PER_W = N // NW
CH = 256
NCHUNK = PER_W // CH
PAIRS = NCHUNK // 2


def _sc_gather(idx_flat, table):
    mesh = plsc.VectorSubcoreMesh(core_axis_name="c", subcore_axis_name="s")

    @functools.partial(
        pl.kernel,
        out_type=jax.ShapeDtypeStruct((N, EMBED), jnp.float32),
        mesh=mesh,
        scratch_types=[
            pltpu.VMEM((CH,), jnp.int32),
            pltpu.VMEM((CH,), jnp.int32),
            pltpu.VMEM((CH, EMBED), jnp.float32),
            pltpu.VMEM((CH, EMBED), jnp.float32),
            pltpu.VMEM((CH * EMBED,), jnp.int32),
            pltpu.SemaphoreType.DMA,
            pltpu.SemaphoreType.DMA,
            pltpu.SemaphoreType.DMA,
            pltpu.SemaphoreType.DMA,
            pltpu.SemaphoreType.DMA,
            pltpu.SemaphoreType.DMA,
        ],
    )
    def k(idx_hbm, table_hbm, out_hbm, idx0, idx1, rows0, rows1, dummy_v,
          isem0, isem1, gsem0, gsem1, wsem0, wsem1):
        wid = lax.axis_index("s") * NC + lax.axis_index("c")
        base = wid * PER_W
        idx_s = (idx0, idx1)
        rows_v = (rows0, rows1)
        isem = (isem0, isem1)
        gsem = (gsem0, gsem1)
        wsem = (wsem0, wsem1)

        def start_idx(c, b):
            pltpu.async_copy(
                idx_hbm.at[pl.ds(base + c * CH, CH)], idx_s[b], isem[b])

        def wait_idx(b):
            pltpu.make_async_copy(
                idx_hbm.at[pl.ds(0, CH)], idx_s[b], isem[b]).wait()

        def enqueue_rows(b):
            @plsc.parallel_loop(0, CH // 16, step=1, unroll=4)
            def _(g):
                vec = idx_s[b][pl.ds(g * 16, 16)]
                for j in range(16):
                    i = vec[j]
                    pltpu.async_copy(table_hbm.at[i],
                                     rows_v[b].at[g * 16 + j], gsem[b])

        def drain_rows(b):
            pltpu.make_async_copy(
                idx_hbm.at[pl.ds(0, CH * EMBED)], dummy_v, gsem[b]).wait()

        def start_wb(c, b):
            pltpu.async_copy(
                rows_v[b], out_hbm.at[pl.ds(base + c * CH, CH)], wsem[b])

        def wait_wb(b):
            pltpu.make_async_copy(
                rows_v[b], out_hbm.at[pl.ds(0, CH)], wsem[b]).wait()

        start_idx(0, 0)
        start_idx(1, 1)
        wait_idx(0)
        enqueue_rows(0)

        drain_rows(0)
        start_wb(0, 0)
        start_idx(2, 0)
        wait_idx(1)
        enqueue_rows(1)

        drain_rows(1)
        start_wb(1, 1)
        start_idx(3, 1)
        wait_wb(0)
        wait_idx(0)
        enqueue_rows(0)

        def pair_body(p, carry):
            c0 = 2 * p
            drain_rows(0)
            start_wb(c0, 0)
            start_idx(c0 + 2, 0)
            wait_wb(1)
            wait_idx(1)
            enqueue_rows(1)

            drain_rows(1)
            start_wb(c0 + 1, 1)
            start_idx(c0 + 3, 1)
            wait_wb(0)
            wait_idx(0)
            enqueue_rows(0)
            return carry

        lax.fori_loop(1, PAIRS - 1, pair_body, 0)

        drain_rows(0)
        start_wb(NCHUNK - 2, 0)
        wait_wb(1)
        wait_idx(1)
        enqueue_rows(1)

        drain_rows(1)
        start_wb(NCHUNK - 1, 1)

        wait_wb(0)
        wait_wb(1)

    return k(idx_flat, table)


def _mask_body(t_ref, m_ref):
    m_ref[...] = t_ref[...] == 0


def _tc_mask(term):
    blk = 256
    return pl.pallas_call(
        _mask_body,
        out_shape=jax.ShapeDtypeStruct((BATCH, HIST), jnp.bool_),
        in_specs=[pl.BlockSpec((blk, HIST), lambda i: (i, 0))],
        out_specs=pl.BlockSpec((blk, HIST), lambda i: (i, 0)),
        grid=(BATCH // blk,),
    )(term)


def kernel(term, table):
    idx_flat = term.reshape(N)
    rows = _sc_gather(idx_flat, table)
    embedded = rows.reshape(BATCH, HIST, EMBED)
    mask = _tc_mask(term)
    return (embedded, mask)


# speedup vs baseline: 1.6375x; 1.0014x over previous
import functools

import jax
import jax.numpy as jnp
from jax import lax
from jax.experimental import pallas as pl
from jax.experimental.pallas import tpu as pltpu
from jax.experimental.pallas import tpu_sc as plsc

BATCH = 4096
HIST = 200
EMBED = 64
N = BATCH * HIST
NC, NS = 2, 16
NW = NC * NS
PER_W = N // NW
CH = 256
NCHUNK = PER_W // CH
PAIRS = NCHUNK // 2


def _sc_gather(idx_flat, table):
    mesh = plsc.VectorSubcoreMesh(core_axis_name="c", subcore_axis_name="s")

    @functools.partial(
        pl.kernel,
        out_type=jax.ShapeDtypeStruct((N, EMBED), jnp.float32),
        mesh=mesh,
        scratch_types=[
            pltpu.VMEM((CH,), jnp.int32),
            pltpu.VMEM((CH,), jnp.int32),
            pltpu.VMEM((CH, EMBED), jnp.float32),
            pltpu.VMEM((CH, EMBED), jnp.float32),
            pltpu.VMEM((CH * EMBED,), jnp.int32),
            pltpu.SemaphoreType.DMA,
            pltpu.SemaphoreType.DMA,
            pltpu.SemaphoreType.DMA,
            pltpu.SemaphoreType.DMA,
            pltpu.SemaphoreType.DMA,
            pltpu.SemaphoreType.DMA,
        ],
    )
    def k(idx_hbm, table_hbm, out_hbm, idx0, idx1, rows0, rows1, dummy_v,
          isem0, isem1, gsem0, gsem1, wsem0, wsem1):
        wid = lax.axis_index("s") * NC + lax.axis_index("c")
        base = wid * PER_W
        idx_s = (idx0, idx1)
        rows_v = (rows0, rows1)
        isem = (isem0, isem1)
        gsem = (gsem0, gsem1)
        wsem = (wsem0, wsem1)

        def start_idx(c, b):
            pltpu.async_copy(
                idx_hbm.at[pl.ds(base + c * CH, CH)], idx_s[b], isem[b])

        def wait_idx(b):
            pltpu.make_async_copy(
                idx_hbm.at[pl.ds(0, CH)], idx_s[b], isem[b]).wait()

        def enqueue_rows(b):
            def group(g, carry):
                vec = idx_s[b][pl.ds(g * 16, 16)]
                for j in range(16):
                    i = vec[j]
                    pltpu.async_copy(table_hbm.at[i],
                                     rows_v[b].at[g * 16 + j], gsem[b])
                return carry

            lax.fori_loop(0, CH // 16, group, 0)

        def drain_rows(b):
            pltpu.make_async_copy(
                idx_hbm.at[pl.ds(0, CH * EMBED)], dummy_v, gsem[b]).wait()

        def start_wb(c, b):
            pltpu.async_copy(
                rows_v[b], out_hbm.at[pl.ds(base + c * CH, CH)], wsem[b])

        def wait_wb(b):
            pltpu.make_async_copy(
                rows_v[b], out_hbm.at[pl.ds(0, CH)], wsem[b]).wait()

        start_idx(0, 0)
        start_idx(1, 1)
        wait_idx(0)
        enqueue_rows(0)

        drain_rows(0)
        start_wb(0, 0)
        start_idx(2, 0)
        wait_idx(1)
        enqueue_rows(1)

        drain_rows(1)
        start_wb(1, 1)
        start_idx(3, 1)
        wait_wb(0)
        wait_idx(0)
        enqueue_rows(0)

        def pair_body(p, carry):
            c0 = 2 * p
            drain_rows(0)
            start_wb(c0, 0)
            start_idx(c0 + 2, 0)
            wait_wb(1)
            wait_idx(1)
            enqueue_rows(1)

            drain_rows(1)
            start_wb(c0 + 1, 1)
            start_idx(c0 + 3, 1)
            wait_wb(0)
            wait_idx(0)
            enqueue_rows(0)
            return carry

        lax.fori_loop(1, PAIRS - 1, pair_body, 0)

        drain_rows(0)
        start_wb(NCHUNK - 2, 0)
        wait_wb(1)
        wait_idx(1)
        enqueue_rows(1)

        drain_rows(1)
        start_wb(NCHUNK - 1, 1)

        wait_wb(0)
        wait_wb(1)

    return k(idx_flat, table)


def _mask_body(t_ref, m_ref):
    m_ref[...] = t_ref[...] == 0


def _tc_mask(term):
    blk = 256
    return pl.pallas_call(
        _mask_body,
        out_shape=jax.ShapeDtypeStruct((BATCH, HIST), jnp.bool_),
        in_specs=[pl.BlockSpec((blk, HIST), lambda i: (i, 0))],
        out_specs=pl.BlockSpec((blk, HIST), lambda i: (i, 0)),
        grid=(BATCH // blk,),
    )(term)


def kernel(term, table):
    idx_flat = term.reshape(N)
    rows = _sc_gather(idx_flat, table)
    embedded = rows.reshape(BATCH, HIST, EMBED)
    mask = _tc_mask(term)
    return (embedded, mask)
